# baseline (device time: 11737 ns/iter reference)
import jax
import jax.numpy as jnp
from jax import lax
from jax.experimental import pallas as pl
from jax.experimental.pallas import tpu as pltpu

N_DEV = 4
EPS = 1e-5


def kernel(x, Wp):
    b, h_loc, w, c = x.shape
    c_out = Wp.shape[1]
    n_global = N_DEV * h_loc * w
    h2 = h_loc // 2
    n_half = h2 * w

    def body(
        x_hbm, wp_hbm, out_hbm,
        x_vmem, wp_vmem, o_vmem, stats_ref,
        cp_sems, send_sems, recv_sems,
    ):
        my_pos = lax.axis_index("i")

        cp_x0 = pltpu.make_async_copy(
            x_hbm.at[:, 0:h2], x_vmem.at[:, 0:h2], cp_sems.at[0]
        )
        cp_x1 = pltpu.make_async_copy(
            x_hbm.at[:, h2:h_loc], x_vmem.at[:, h2:h_loc], cp_sems.at[1]
        )
        cp_w = pltpu.make_async_copy(wp_hbm, wp_vmem, cp_sems.at[2])
        cp_x0.start()
        cp_x1.start()
        cp_w.start()

        barrier_sem = pltpu.get_barrier_semaphore()
        for k in (2, 1, 3):
            peer = (my_pos + k) % N_DEV
            pl.semaphore_signal(
                barrier_sem, inc=1,
                device_id=(peer,), device_id_type=pl.DeviceIdType.MESH,
            )

        rows = lax.broadcasted_iota(jnp.int32, (b, b * n_half), 0)
        cols = lax.broadcasted_iota(jnp.int32, (b, b * n_half), 1)
        sel = (cols // n_half == rows).astype(jnp.bfloat16)

        def half_stats(x_bf):
            flat = x_bf.reshape(b * n_half, c)
            s1 = jnp.dot(sel, flat, preferred_element_type=jnp.float32)
            s2 = jnp.dot(sel, flat * flat, preferred_element_type=jnp.float32)
            return s1, s2

        cp_x0.wait()
        x0_bf = x_vmem[:, 0:h2].astype(jnp.bfloat16)
        s1a, s2a = half_stats(x0_bf)
        cp_x1.wait()
        x1_bf = x_vmem[:, h2:h_loc].astype(jnp.bfloat16)
        s1b, s2b = half_stats(x1_bf)
        stats_ref[0, :, :] = jnp.concatenate([s1a + s1b, s2a + s2b], axis=0)

        pl.semaphore_wait(barrier_sem, N_DEV - 1)

        rdmas = []
        for k in (2, 1, 3):
            peer = (my_pos + k) % N_DEV
            rdma = pltpu.make_async_remote_copy(
                src_ref=stats_ref.at[0],
                dst_ref=stats_ref.at[k],
                send_sem=send_sems.at[k - 1],
                recv_sem=recv_sems.at[k - 1],
                device_id=(peer,),
                device_id_type=pl.DeviceIdType.MESH,
            )
            rdma.start()
            rdmas.append(rdma)

        cp_w.wait()
        wp = wp_vmem[...].astype(jnp.bfloat16)

        for rdma in rdmas:
            rdma.wait_send()
        for rdma in rdmas:
            rdma.wait_recv()

        tot = (
            stats_ref[0, :, :] + stats_ref[1, :, :]
            + stats_ref[2, :, :] + stats_ref[3, :, :]
        )
        mean = tot[:b, :] / n_global
        var = tot[b:, :] / n_global - mean * mean
        inv = lax.rsqrt(var + EPS)
        mean_bf = mean.astype(jnp.bfloat16).reshape(b, 1, 1, c)
        inv_bf = inv.astype(jnp.bfloat16).reshape(b, 1, 1, c)

        def half_out(x_bf):
            h = (x_bf - mean_bf) * inv_bf
            a = (h * jax.nn.sigmoid(h)).reshape(b * n_half, c)
            o = jnp.dot(a, wp, preferred_element_type=jnp.float32)
            return o.reshape(b, h2, w, c_out).astype(jnp.bfloat16)

        o_vmem[:, 0:h2] = half_out(x0_bf)
        cp_o0 = pltpu.make_async_copy(
            o_vmem.at[:, 0:h2], out_hbm.at[:, 0:h2], cp_sems.at[3]
        )
        cp_o0.start()
        o_vmem[:, h2:h_loc] = half_out(x1_bf)
        cp_o1 = pltpu.make_async_copy(
            o_vmem.at[:, h2:h_loc], out_hbm.at[:, h2:h_loc], cp_sems.at[4]
        )
        cp_o1.start()
        cp_o0.wait()
        cp_o1.wait()

    return pl.pallas_call(
        body,
        out_shape=jax.ShapeDtypeStruct((b, h_loc, w, c_out), jnp.bfloat16),
        in_specs=[
            pl.BlockSpec(memory_space=pl.ANY),
            pl.BlockSpec(memory_space=pl.ANY),
        ],
        out_specs=pl.BlockSpec(memory_space=pl.ANY),
        scratch_shapes=[
            pltpu.VMEM((b, h_loc, w, c), jnp.float32),
            pltpu.VMEM((c, c_out), jnp.float32),
            pltpu.VMEM((b, h_loc, w, c_out), jnp.bfloat16),
            pltpu.VMEM((N_DEV, 2 * b, c), jnp.float32),
            pltpu.SemaphoreType.DMA((5,)),
            pltpu.SemaphoreType.DMA((N_DEV - 1,)),
            pltpu.SemaphoreType.DMA((N_DEV - 1,)),
        ],
        compiler_params=pltpu.CompilerParams(collective_id=0),
    )(x, Wp)


# device time: 11099 ns/iter; 1.0575x vs baseline; 1.0575x over previous
import jax
import jax.numpy as jnp
from jax import lax
from jax.experimental import pallas as pl
from jax.experimental.pallas import tpu as pltpu

N_DEV = 4
EPS = 1e-5


def kernel(x, Wp):
    b, h_loc, w, c = x.shape
    c_out = Wp.shape[1]
    n_global = N_DEV * h_loc * w

    def body(x_ref, wp_ref, out_ref, stats_ref, send_sems, recv_sems):
        my_pos = lax.axis_index("i")

        barrier_sem = pltpu.get_barrier_semaphore()
        for k in (2, 1, 3):
            peer = (my_pos + k) % N_DEV
            pl.semaphore_signal(
                barrier_sem, inc=1,
                device_id=(peer,), device_id_type=pl.DeviceIdType.MESH,
            )

        n_loc = h_loc * w
        x_bf = x_ref[...].astype(jnp.bfloat16)
        x_flat = x_bf.reshape(b * n_loc, c)
        rows = lax.broadcasted_iota(jnp.int32, (b, b * n_loc), 0)
        cols = lax.broadcasted_iota(jnp.int32, (b, b * n_loc), 1)
        sel = (cols // n_loc == rows).astype(jnp.bfloat16)
        s1 = jnp.dot(sel, x_flat, preferred_element_type=jnp.float32)
        s2 = jnp.dot(sel, x_flat * x_flat, preferred_element_type=jnp.float32)
        stats_ref[0, :, :] = jnp.concatenate([s1, s2], axis=0)

        pl.semaphore_wait(barrier_sem, N_DEV - 1)

        rdmas = []
        for k in (2, 1, 3):
            peer = (my_pos + k) % N_DEV
            rdma = pltpu.make_async_remote_copy(
                src_ref=stats_ref.at[0],
                dst_ref=stats_ref.at[k],
                send_sem=send_sems.at[k - 1],
                recv_sem=recv_sems.at[k - 1],
                device_id=(peer,),
                device_id_type=pl.DeviceIdType.MESH,
            )
            rdma.start()
            rdmas.append(rdma)

        wp = wp_ref[...].astype(jnp.bfloat16)

        for rdma in rdmas:
            rdma.wait_send()
        for rdma in rdmas:
            rdma.wait_recv()

        tot = (
            stats_ref[0, :, :] + stats_ref[1, :, :]
            + stats_ref[2, :, :] + stats_ref[3, :, :]
        )
        mean = tot[:b, :] / n_global
        var = tot[b:, :] / n_global - mean * mean
        inv = lax.rsqrt(var + EPS)
        mean_bf = mean.astype(jnp.bfloat16).reshape(b, 1, 1, c)
        inv_bf = inv.astype(jnp.bfloat16).reshape(b, 1, 1, c)
        h = (x_bf - mean_bf) * inv_bf
        a = (h * jax.nn.sigmoid(h)).reshape(b * n_loc, c)
        o = jnp.dot(a, wp, preferred_element_type=jnp.float32)
        out_ref[...] = o.reshape(b, h_loc, w, c_out).astype(jnp.bfloat16)

    return pl.pallas_call(
        body,
        out_shape=jax.ShapeDtypeStruct((b, h_loc, w, c_out), jnp.bfloat16),
        in_specs=[
            pl.BlockSpec(memory_space=pltpu.VMEM),
            pl.BlockSpec(memory_space=pltpu.VMEM),
        ],
        out_specs=pl.BlockSpec(memory_space=pltpu.VMEM),
        scratch_shapes=[
            pltpu.VMEM((N_DEV, 2 * b, c), jnp.float32),
            pltpu.SemaphoreType.DMA((N_DEV - 1,)),
            pltpu.SemaphoreType.DMA((N_DEV - 1,)),
        ],
        compiler_params=pltpu.CompilerParams(collective_id=0),
    )(x, Wp)
